# baseline (speedup 1.0000x reference)
import jax
import jax.numpy as jnp
from jax.experimental import pallas as pl
from jax.experimental.pallas import tpu as pltpu


_CHUNKS = (
    (0, 256), (256, 256), (512, 512), (1024, 1024),
    (2048, 2048), (4096, 2048),
    (6144, 1024), (7168, 512), (7680, 256), (7936, 256),
)
_TPIECES = ((0, 256), (256, 256), (512, 512), (1024, 1024))


def _body(xf, tab, out, xbuf, tbuf, load_sem, store_sem, tab_sem):
    for p, (ts, tl) in enumerate(_TPIECES):
        pltpu.make_async_copy(
            tab.at[pl.ds(ts, tl)], tbuf.at[pl.ds(ts, tl)], tab_sem.at[p]).start()
        cs, cl = _CHUNKS[p]
        pltpu.make_async_copy(
            xf.at[pl.ds(cs, cl)], xbuf.at[pl.ds(cs, cl)], load_sem.at[p]).start()
    for k in range(len(_TPIECES), len(_CHUNKS)):
        cs, cl = _CHUNKS[k]
        pltpu.make_async_copy(
            xf.at[pl.ds(cs, cl)], xbuf.at[pl.ds(cs, cl)], load_sem.at[k]).start()

    for k, (cs, cl) in enumerate(_CHUNKS):
        if k < len(_TPIECES):
            ts, tl = _TPIECES[k]
            pltpu.make_async_copy(
                tab.at[pl.ds(ts, tl)], tbuf.at[pl.ds(ts, tl)],
                tab_sem.at[k]).wait()
            t = tbuf[pl.ds(ts, tl)]
            norm = jnp.sqrt(jnp.sum(t * t, axis=-1, keepdims=True))
            scale = jnp.where(norm > 1.0, 1.0 / (norm + 1e-7), 1.0)
            tbuf[pl.ds(ts, tl)] = t * scale
        pltpu.make_async_copy(
            xf.at[pl.ds(cs, cl)], xbuf.at[pl.ds(cs, cl)], load_sem.at[k]).wait()
        S = tab.shape[0]
        xbuf[pl.ds(cs, cl)] = xbuf[pl.ds(cs, cl)] + tbuf[pl.ds(cs % S, cl)]
        pltpu.make_async_copy(
            xbuf.at[pl.ds(cs, cl)], out.at[pl.ds(cs, cl)], store_sem.at[k]).start()

    for k, (cs, cl) in enumerate(_CHUNKS):
        pltpu.make_async_copy(
            xbuf.at[pl.ds(cs, cl)], out.at[pl.ds(cs, cl)], store_sem.at[k]).wait()


def kernel(x, table):
    B, S, D = x.shape
    xf = x.reshape(B * S, D)
    out = pl.pallas_call(
        _body,
        grid=(1,),
        in_specs=[
            pl.BlockSpec(memory_space=pl.ANY),
            pl.BlockSpec(memory_space=pl.ANY),
        ],
        out_specs=pl.BlockSpec(memory_space=pl.ANY),
        out_shape=jax.ShapeDtypeStruct((B * S, D), x.dtype),
        scratch_shapes=[
            pltpu.VMEM((B * S, D), jnp.float32),
            pltpu.VMEM((S, D), jnp.float32),
            pltpu.SemaphoreType.DMA((len(_CHUNKS),)),
            pltpu.SemaphoreType.DMA((len(_CHUNKS),)),
            pltpu.SemaphoreType.DMA((len(_TPIECES),)),
        ],
        compiler_params=pltpu.CompilerParams(
            dimension_semantics=("arbitrary",)),
    )(xf, table)
    return out.reshape(B, S, D)


# speedup vs baseline: 1.0182x; 1.0182x over previous
import jax
import jax.numpy as jnp
from jax.experimental import pallas as pl
from jax.experimental.pallas import tpu as pltpu


_C = 2048
_NB = 3
_K = 2


def _body(xf, tab, out, xin, xout, tbuf, load_sem, store_sem, tab_sem):
    i = pl.program_id(0)
    T = pl.num_programs(0)
    S = tab.shape[0]

    def start_load(t):
        s = jax.lax.rem(t, _NB)
        pltpu.make_async_copy(
            xf.at[pl.ds(t * _C, _C)], xin.at[s], load_sem.at[s]).start()

    @pl.when(i == 0)
    def _prologue():
        pltpu.make_async_copy(tab, tbuf, tab_sem).start()
        for t in range(_K):
            start_load(t)
        pltpu.make_async_copy(tab, tbuf, tab_sem).wait()
        tb = tbuf[...]
        norm = jnp.sqrt(jnp.sum(tb * tb, axis=-1, keepdims=True))
        scale = jnp.where(norm > 1.0, 1.0 / (norm + 1e-7), 1.0)
        tbuf[...] = tb * scale

    s = jax.lax.rem(i, _NB)

    @pl.when(i >= _NB)
    def _retire_prev_store():
        pltpu.make_async_copy(
            xout.at[s], out.at[pl.ds((i - _NB) * _C, _C)],
            store_sem.at[s]).wait()

    pltpu.make_async_copy(
        xf.at[pl.ds(i * _C, _C)], xin.at[s], load_sem.at[s]).wait()
    trow = jax.lax.rem(i * _C, S)
    xout[s] = xin[s] + tbuf[pl.ds(trow, _C)]
    pltpu.make_async_copy(
        xout.at[s], out.at[pl.ds(i * _C, _C)], store_sem.at[s]).start()

    @pl.when(i + _K < T)
    def _prefetch():
        start_load(i + _K)

    @pl.when(i == T - 1)
    def _epilogue():
        for d in range(_NB):
            t = T - _NB + d
            if t >= 0:
                ss = t % _NB
                pltpu.make_async_copy(
                    xout.at[ss], out.at[pl.ds(t * _C, _C)],
                    store_sem.at[ss]).wait()


def kernel(x, table):
    B, S, D = x.shape
    xf = x.reshape(B * S, D)
    T = (B * S) // _C
    out = pl.pallas_call(
        _body,
        grid=(T,),
        in_specs=[
            pl.BlockSpec(memory_space=pl.ANY),
            pl.BlockSpec(memory_space=pl.ANY),
        ],
        out_specs=pl.BlockSpec(memory_space=pl.ANY),
        out_shape=jax.ShapeDtypeStruct((B * S, D), x.dtype),
        scratch_shapes=[
            pltpu.VMEM((_NB, _C, D), jnp.float32),
            pltpu.VMEM((_NB, _C, D), jnp.float32),
            pltpu.VMEM((S, D), jnp.float32),
            pltpu.SemaphoreType.DMA((_NB,)),
            pltpu.SemaphoreType.DMA((_NB,)),
            pltpu.SemaphoreType.DMA,
        ],
        compiler_params=pltpu.CompilerParams(
            dimension_semantics=("arbitrary",)),
    )(xf, table)
    return out.reshape(B, S, D)


# speedup vs baseline: 1.0553x; 1.0364x over previous
import jax
import jax.numpy as jnp
from jax.experimental import pallas as pl
from jax.experimental.pallas import tpu as pltpu


_C = 2048
_NB = 3
_K = 3


def _body(xf, tab, out, xin, xout, tbuf, load_sem, store_sem, tab_sem):
    i = pl.program_id(0)
    T = pl.num_programs(0)
    S = tab.shape[0]

    def start_load(t):
        s = jax.lax.rem(t, _NB)
        pltpu.make_async_copy(
            xf.at[pl.ds(t * _C, _C)], xin.at[s], load_sem.at[s]).start()

    @pl.when(i == 0)
    def _prologue():
        pltpu.make_async_copy(tab, tbuf, tab_sem).start()
        for t in range(_K):
            start_load(t)
        pltpu.make_async_copy(tab, tbuf, tab_sem).wait()
        tb = tbuf[...]
        norm = jnp.sqrt(jnp.sum(tb * tb, axis=-1, keepdims=True))
        scale = jnp.where(norm > 1.0, 1.0 / (norm + 1e-7), 1.0)
        tbuf[...] = tb * scale

    s = jax.lax.rem(i, _NB)

    @pl.when(i >= _NB)
    def _retire_prev_store():
        pltpu.make_async_copy(
            xout.at[s], out.at[pl.ds((i - _NB) * _C, _C)],
            store_sem.at[s]).wait()

    pltpu.make_async_copy(
        xf.at[pl.ds(i * _C, _C)], xin.at[s], load_sem.at[s]).wait()
    trow = jax.lax.rem(i * _C, S)
    xout[s] = xin[s] + tbuf[pl.ds(trow, _C)]
    pltpu.make_async_copy(
        xout.at[s], out.at[pl.ds(i * _C, _C)], store_sem.at[s]).start()

    @pl.when(i + _K < T)
    def _prefetch():
        start_load(i + _K)

    @pl.when(i == T - 1)
    def _epilogue():
        for d in range(_NB):
            t = T - _NB + d
            if t >= 0:
                ss = t % _NB
                pltpu.make_async_copy(
                    xout.at[ss], out.at[pl.ds(t * _C, _C)],
                    store_sem.at[ss]).wait()


def kernel(x, table):
    B, S, D = x.shape
    xf = x.reshape(B * S, D)
    T = (B * S) // _C
    out = pl.pallas_call(
        _body,
        grid=(T,),
        in_specs=[
            pl.BlockSpec(memory_space=pl.ANY),
            pl.BlockSpec(memory_space=pl.ANY),
        ],
        out_specs=pl.BlockSpec(memory_space=pl.ANY),
        out_shape=jax.ShapeDtypeStruct((B * S, D), x.dtype),
        scratch_shapes=[
            pltpu.VMEM((_NB, _C, D), jnp.float32),
            pltpu.VMEM((_NB, _C, D), jnp.float32),
            pltpu.VMEM((S, D), jnp.float32),
            pltpu.SemaphoreType.DMA((_NB,)),
            pltpu.SemaphoreType.DMA((_NB,)),
            pltpu.SemaphoreType.DMA,
        ],
        compiler_params=pltpu.CompilerParams(
            dimension_semantics=("arbitrary",)),
    )(xf, table)
    return out.reshape(B, S, D)
